# baseline (speedup 1.0000x reference)
import functools

import jax
import jax.numpy as jnp
from jax import lax
from jax.experimental import pallas as pl
from jax.experimental.pallas import tpu as pltpu
from jax.experimental.pallas import tpu_sc as plsc

_VOCAB = 1000
_VOCAB_PAD = 1024
_L = 200
_B = 16384
_COLS_PER_W = 7
_MAX_C0 = _L - _COLS_PER_W

_NEG = float(jnp.finfo(jnp.float32).min)


def _presence_body(xflat_hbm, out_hbm, xv, pres):
    wid = lax.axis_index("s") * 2 + lax.axis_index("c")
    c0 = jnp.minimum(wid * _COLS_PER_W, _MAX_C0)

    src_off = pl.multiple_of(c0 * _B, 8)
    pltpu.sync_copy(xflat_hbm.at[pl.ds(src_off, _COLS_PER_W * _B)], xv)

    zeros16 = jnp.zeros((16,), jnp.float32)

    @plsc.parallel_loop(0, _COLS_PER_W * _VOCAB_PAD // 16, unroll=8)
    def _(i):
        pres[pl.ds(i * 16, 16)] = zeros16

    ones16 = jnp.ones((16,), jnp.float32)
    pad16 = jnp.where(lax.iota(jnp.int32, 16) < 8, 0.0, 1.0)
    for cl in range(_COLS_PER_W):
        pres[pl.ds(cl * _VOCAB_PAD + 992, 16)] = pad16
        pres[pl.ds(cl * _VOCAB_PAD + 1008, 16)] = ones16

    ones16 = jnp.ones((16,), jnp.float32)

    @plsc.parallel_loop(0, _B // 32, unroll=2)
    def _(i):
        base = i * 32
        for cl in range(_COLS_PER_W):
            v2 = xv[pl.ds(cl * _B + base, 32)]
            va, vb = plsc.unpack(v2, format=plsc.PackFormat.INTERLEAVED,
                                 preferred_element_type=jnp.int32)
            plsc.store_scatter(pres, [va + cl * _VOCAB_PAD], ones16)
            plsc.store_scatter(pres, [vb + cl * _VOCAB_PAD], ones16)

    dst_off = pl.multiple_of(c0 * _VOCAB_PAD, 8)
    pltpu.sync_copy(pres, out_hbm.at[pl.ds(dst_off, _COLS_PER_W * _VOCAB_PAD)])


@functools.cache
def _presence_sc():
    mesh = plsc.VectorSubcoreMesh(core_axis_name="c", subcore_axis_name="s")
    return pl.kernel(
        _presence_body,
        out_type=jax.ShapeDtypeStruct((_L * _VOCAB_PAD,), jnp.float32),
        mesh=mesh,
        scratch_types=[
            pltpu.VMEM((_COLS_PER_W * _B,), jnp.int16),
            pltpu.VMEM((_COLS_PER_W * _VOCAB_PAD,), jnp.float32),
        ],
        compiler_params=pltpu.CompilerParams(needs_layout_passes=False),
    )


def _mlp_tc(p_ref, emb_ref, w1_ref, b1_ref, w2_ref, b2_ref, w3_ref, b3_ref,
            out_ref, pT_s):
    pmin = jnp.inf
    for k in range(_VOCAB_PAD // 128):
        pmin = jnp.minimum(pmin, jnp.min(p_ref[:, k * 128:(k + 1) * 128]))

    def fast(_):
        colmax = jnp.max(emb_ref[...], axis=0)
        return jnp.broadcast_to(colmax[None, :], (_L, 128))

    def slow(_):
        for k in range(_VOCAB_PAD // 128):
            pblk = p_ref[:, k * 128:(k + 1) * 128]
            pT_s[k * 128:(k + 1) * 128, :] = jnp.swapaxes(pblk, 0, 1)

        def step(i, m):
            off = pl.multiple_of(i * 8, 8)
            pblk = pT_s[pl.ds(off, 8), :]
            eblk = emb_ref[pl.ds(off, 8), :]
            masked = jnp.where(pblk[:, :, None] > 0, eblk[:, None, :], _NEG)
            return jnp.maximum(m, jnp.max(masked, axis=0))

        minit = jnp.full((_L, 128), _NEG, jnp.float32)
        return lax.fori_loop(0, _VOCAB // 8, step, minit)

    m = lax.cond(pmin > 0, fast, slow, 0)

    h = jnp.maximum(
        jnp.dot(m, w1_ref[...], preferred_element_type=jnp.float32)
        + b1_ref[...], 0.0)
    h = jnp.maximum(
        jnp.dot(h, w2_ref[...], preferred_element_type=jnp.float32)
        + b2_ref[...], 0.0)
    out_ref[...] = (
        jnp.dot(h, w3_ref[...], preferred_element_type=jnp.float32)
        + b3_ref[...])


_mlp_call = pl.pallas_call(
    _mlp_tc,
    out_shape=jax.ShapeDtypeStruct((_L, 100), jnp.float32),
    scratch_shapes=[pltpu.VMEM((_VOCAB_PAD, _L), jnp.float32)],
)


def kernel(x, emb, W1, b1, W2, b2, W3, b3):
    xT = jnp.transpose(x.astype(jnp.int16))
    pres = _presence_sc()(xT.reshape(-1)).reshape(_L, _VOCAB_PAD)
    embp = jnp.concatenate(
        [emb, jnp.full((_VOCAB_PAD - _VOCAB, 128), _NEG, jnp.float32)])
    return _mlp_call(pres, embp, W1, b1.reshape(1, 128), W2,
                     b2.reshape(1, 128), W3, b3.reshape(1, 100))


# speedup vs baseline: 1.7078x; 1.7078x over previous
import functools

import jax
import jax.numpy as jnp
from jax import lax
from jax.experimental import pallas as pl
from jax.experimental.pallas import tpu as pltpu
from jax.experimental.pallas import tpu_sc as plsc

_VOCAB = 1000
_VOCAB_PAD = 1024
_L = 200
_B = 16384
_COLS_PER_W = 7
_MAX_C0 = _L - _COLS_PER_W

_NEG = float(jnp.finfo(jnp.float32).min)


def _presence_body(xflat_hbm, out_hbm, xva, xvb, pres, sem_a, sem_b):
    wid = lax.axis_index("s") * 2 + lax.axis_index("c")
    c0 = jnp.minimum(wid * _COLS_PER_W, _MAX_C0)

    off_a = pl.multiple_of(c0 * _B, 8)
    cp_a = pltpu.async_copy(xflat_hbm.at[pl.ds(off_a, 4 * _B)], xva, sem_a)
    off_b = pl.multiple_of((c0 + 4) * _B, 8)
    cp_b = pltpu.async_copy(xflat_hbm.at[pl.ds(off_b, 3 * _B)], xvb, sem_b)

    zeros16 = jnp.zeros((16,), jnp.float32)

    @plsc.parallel_loop(0, _COLS_PER_W * _VOCAB_PAD // 16, unroll=8)
    def _(i):
        pres[pl.ds(i * 16, 16)] = zeros16

    ones16 = jnp.ones((16,), jnp.float32)
    pad16 = jnp.where(lax.iota(jnp.int32, 16) < 8, 0.0, 1.0)
    for cl in range(_COLS_PER_W):
        pres[pl.ds(cl * _VOCAB_PAD + 992, 16)] = pad16
        pres[pl.ds(cl * _VOCAB_PAD + 1008, 16)] = ones16

    ones16 = jnp.ones((16,), jnp.float32)
    cp_a.wait()

    @plsc.parallel_loop(0, _B // 16, unroll=4)
    def _(i):
        base = i * 16
        for cl in range(4):
            vals = xva[pl.ds(cl * _B + base, 16)]
            plsc.store_scatter(pres, [vals + cl * _VOCAB_PAD], ones16)

    cp_b.wait()

    @plsc.parallel_loop(0, _B // 16, unroll=4)
    def _(i):
        base = i * 16
        for cl in range(3):
            vals = xvb[pl.ds(cl * _B + base, 16)]
            plsc.store_scatter(pres, [vals + (cl + 4) * _VOCAB_PAD], ones16)

    dst_off = pl.multiple_of(c0 * _VOCAB_PAD, 8)
    pltpu.sync_copy(pres, out_hbm.at[pl.ds(dst_off, _COLS_PER_W * _VOCAB_PAD)])


@functools.cache
def _presence_sc():
    mesh = plsc.VectorSubcoreMesh(core_axis_name="c", subcore_axis_name="s")
    return pl.kernel(
        _presence_body,
        out_type=jax.ShapeDtypeStruct((_L * _VOCAB_PAD,), jnp.float32),
        mesh=mesh,
        scratch_types=[
            pltpu.VMEM((4 * _B,), jnp.int32),
            pltpu.VMEM((3 * _B,), jnp.int32),
            pltpu.VMEM((_COLS_PER_W * _VOCAB_PAD,), jnp.float32),
            pltpu.SemaphoreType.DMA,
            pltpu.SemaphoreType.DMA,
        ],
        compiler_params=pltpu.CompilerParams(needs_layout_passes=False),
    )


def _mlp_tc(p_ref, emb_ref, w1_ref, b1_ref, w2_ref, b2_ref, w3_ref, b3_ref,
            out_ref, pT_s):
    pmin = jnp.inf
    for k in range(_VOCAB_PAD // 128):
        pmin = jnp.minimum(pmin, jnp.min(p_ref[:, k * 128:(k + 1) * 128]))

    def fast(_):
        colmax = jnp.max(emb_ref[...], axis=0)
        return jnp.broadcast_to(colmax[None, :], (_L, 128))

    def slow(_):
        for k in range(_VOCAB_PAD // 128):
            pblk = p_ref[:, k * 128:(k + 1) * 128]
            pT_s[k * 128:(k + 1) * 128, :] = jnp.swapaxes(pblk, 0, 1)

        def step(i, m):
            off = pl.multiple_of(i * 8, 8)
            pblk = pT_s[pl.ds(off, 8), :]
            eblk = emb_ref[pl.ds(off, 8), :]
            masked = jnp.where(pblk[:, :, None] > 0, eblk[:, None, :], _NEG)
            return jnp.maximum(m, jnp.max(masked, axis=0))

        minit = jnp.full((_L, 128), _NEG, jnp.float32)
        return lax.fori_loop(0, _VOCAB // 8, step, minit)

    m = lax.cond(pmin > 0, fast, slow, 0)

    h = jnp.maximum(
        jnp.dot(m, w1_ref[...], preferred_element_type=jnp.float32)
        + b1_ref[...], 0.0)
    h = jnp.maximum(
        jnp.dot(h, w2_ref[...], preferred_element_type=jnp.float32)
        + b2_ref[...], 0.0)
    out_ref[...] = (
        jnp.dot(h, w3_ref[...], preferred_element_type=jnp.float32)
        + b3_ref[...])


_mlp_call = pl.pallas_call(
    _mlp_tc,
    out_shape=jax.ShapeDtypeStruct((_L, 100), jnp.float32),
    scratch_shapes=[pltpu.VMEM((_VOCAB_PAD, _L), jnp.float32)],
)


def kernel(x, emb, W1, b1, W2, b2, W3, b3):
    xT = jnp.transpose(x.astype(jnp.int32))
    pres = _presence_sc()(xT.reshape(-1)).reshape(_L, _VOCAB_PAD)
    embp = jnp.concatenate(
        [emb, jnp.full((_VOCAB_PAD - _VOCAB, 128), _NEG, jnp.float32)])
    return _mlp_call(pres, embp, W1, b1.reshape(1, 128), W2,
                     b2.reshape(1, 128), W3, b3.reshape(1, 100))
